# Initial kernel scaffold; baseline (speedup 1.0000x reference)
import functools
import math

import jax
import jax.numpy as jnp
from jax import lax
from jax.experimental import pallas as pl
from jax.experimental.pallas import tpu as pltpu
from jax.experimental.pallas import tpu_sc as plsc

PI = math.pi
CUTOFF = 10.0
LOG2 = math.log(2.0)

N = 10000
E = 320000
F = 128

NC = 2
NS = 16
EPW = E // (NC * NS)
BE_SC = 200
NB_SC = EPW // BE_SC
RPT = N // NS

BE_TC = 3200


def _ssp(v):
    return jax.nn.softplus(v) - LOG2


def _filter_body(ea_ref, el_ref, w0t_ref, b0_ref, w2t_ref, b2_ref, out_ref):
    a = jnp.dot(ea_ref[...], w0t_ref[...], preferred_element_type=jnp.float32)
    a = _ssp(a + b0_ref[...])
    w = jnp.dot(a, w2t_ref[...], preferred_element_type=jnp.float32) + b2_ref[...]
    el = el_ref[...]
    c = 0.5 * (jnp.cos(el * (PI / CUTOFF)) + 1.0)
    c = c * (el <= CUTOFF).astype(jnp.float32) * (el >= 0.0).astype(jnp.float32)
    out_ref[...] = w * c


def _filter_tc(edge_attr, edge_length, w0t, b0, w2t, b2):
    g = edge_attr.shape[1]
    grid = E // BE_TC
    return pl.pallas_call(
        _filter_body,
        grid=(grid,),
        in_specs=[
            pl.BlockSpec((BE_TC, g), lambda i: (i, 0)),
            pl.BlockSpec((BE_TC, 1), lambda i: (i, 0)),
            pl.BlockSpec((g, F), lambda i: (0, 0)),
            pl.BlockSpec((1, F), lambda i: (0, 0)),
            pl.BlockSpec((F, F), lambda i: (0, 0)),
            pl.BlockSpec((1, F), lambda i: (0, 0)),
        ],
        out_specs=pl.BlockSpec((BE_TC, F), lambda i: (i, 0)),
        out_shape=jax.ShapeDtypeStruct((E, F), jnp.float32),
    )(edge_attr, edge_length, w0t, b0, w2t, b2)


def _lin1_body(x_ref, wt_ref, out_ref):
    out_ref[...] = jnp.dot(x_ref[...], wt_ref[...],
                           preferred_element_type=jnp.float32)


def _lin1_tc(x, lin1t):
    rb = 1250
    return pl.pallas_call(
        _lin1_body,
        grid=(N // rb,),
        in_specs=[
            pl.BlockSpec((rb, F), lambda i: (i, 0)),
            pl.BlockSpec((F, F), lambda i: (0, 0)),
        ],
        out_specs=pl.BlockSpec((rb, F), lambda i: (i, 0)),
        out_shape=jax.ShapeDtypeStruct((N, F), jnp.float32),
    )(x, lin1t)


def _msg_body(h_hbm, w_hbm, src_hbm, dst_hbm, out_hbm,
              src_v, dst_v, rows_v, wv, zbuf, agg_sh, sem):
    c = lax.axis_index("c")
    s = lax.axis_index("s")
    wid = c * NS + s
    wbase = wid * EPW

    def zrow(i, _):
        for j in range(F // 16):
            zbuf[i, pl.ds(j * 16, 16)] = jnp.zeros((16,), jnp.float32)
        return 0
    lax.fori_loop(0, RPT, zrow, 0)
    pltpu.sync_copy(zbuf, agg_sh.at[pl.ds(s * RPT, RPT)])
    plsc.subcore_barrier()

    def block(b, _):
        base = wbase + b * BE_SC
        pltpu.sync_copy(src_hbm.at[pl.ds(base, BE_SC)], src_v)
        pltpu.sync_copy(dst_hbm.at[pl.ds(base, BE_SC)], dst_v)
        gather = pltpu.async_copy(h_hbm.at[src_v], rows_v, sem)
        pltpu.sync_copy(w_hbm.at[pl.ds(base, BE_SC)], wv)
        gather.wait()

        def mul(i, _):
            for j in range(F // 16):
                sl = pl.ds(j * 16, 16)
                rows_v[i, sl] = rows_v[i, sl] * wv[i, sl]
            return 0
        lax.fori_loop(0, BE_SC, mul, 0)

        pltpu.sync_copy(rows_v, agg_sh.at[dst_v], add=True)
        return 0
    lax.fori_loop(0, NB_SC, block, 0)

    plsc.subcore_barrier()
    pltpu.sync_copy(agg_sh.at[pl.ds(s * RPT, RPT)],
                    out_hbm.at[c, pl.ds(s * RPT, RPT)])


def _msg_sc(h, w, src, dst):
    mesh = plsc.VectorSubcoreMesh(core_axis_name="c", subcore_axis_name="s")
    f = pl.kernel(
        _msg_body,
        out_type=jax.ShapeDtypeStruct((NC, N, F), jnp.float32),
        mesh=mesh,
        scratch_types=[
            pltpu.VMEM((BE_SC,), jnp.int32),
            pltpu.VMEM((BE_SC,), jnp.int32),
            pltpu.VMEM((BE_SC, F), jnp.float32),
            pltpu.VMEM((BE_SC, F), jnp.float32),
            pltpu.VMEM((RPT, F), jnp.float32),
            pltpu.VMEM_SHARED((N, F), jnp.float32),
            pltpu.SemaphoreType.DMA,
        ],
    )
    return f(h, w, src, dst)


def _final_body(p_ref, l2t_ref, b2_ref, lt_ref, b_ref, out_ref):
    agg = p_ref[0] + p_ref[1]
    h2 = jnp.dot(agg, l2t_ref[...], preferred_element_type=jnp.float32)
    h2 = _ssp(h2 + b2_ref[...])
    out_ref[...] = jnp.dot(h2, lt_ref[...],
                           preferred_element_type=jnp.float32) + b_ref[...]


def _final_tc(parts, lin2t, b2, lint, b):
    rb = 1250
    out = parts.shape[-1]
    return pl.pallas_call(
        _final_body,
        grid=(N // rb,),
        in_specs=[
            pl.BlockSpec((NC, rb, F), lambda i: (0, i, 0)),
            pl.BlockSpec((F, out), lambda i: (0, 0)),
            pl.BlockSpec((1, out), lambda i: (0, 0)),
            pl.BlockSpec((out, out), lambda i: (0, 0)),
            pl.BlockSpec((1, out), lambda i: (0, 0)),
        ],
        out_specs=pl.BlockSpec((rb, out), lambda i: (i, 0)),
        out_shape=jax.ShapeDtypeStruct((N, out), jnp.float32),
    )(parts, lin2t, b2, lint, b)


@jax.jit
def kernel(x, edge_index, edge_length, edge_attr,
           lin1_w, nnW0_w, nnW0_b, nnW2_w, nnW2_b,
           lin2_w, lin2_b, lin_w, lin_b):
    w = _filter_tc(edge_attr, edge_length[:, None],
                   nnW0_w.T, nnW0_b[None, :], nnW2_w.T, nnW2_b[None, :])
    h = _lin1_tc(x, lin1_w.T)
    src = edge_index[0]
    dst = edge_index[1]
    parts = _msg_sc(h, w, src, dst)
    return _final_tc(parts, lin2_w.T, lin2_b[None, :],
                     lin_w.T, lin_b[None, :])


# speedup vs baseline: 1.3883x; 1.3883x over previous
import functools
import math

import jax
import jax.numpy as jnp
from jax import lax
from jax.experimental import pallas as pl
from jax.experimental.pallas import tpu as pltpu
from jax.experimental.pallas import tpu_sc as plsc

PI = math.pi
CUTOFF = 10.0
LOG2 = math.log(2.0)

N = 10000
E = 320000
F = 128

NC = 2
NS = 16
EPW = E // (NC * NS)
BE_SC = 80
NB_SC = EPW // BE_SC
NPAD = 10240
RPT = NPAD // NS

BE_TC = 3200


def _ssp(v):
    return jax.nn.softplus(v) - LOG2


def _filter_body(ea_ref, el_ref, w0t_ref, b0_ref, w2t_ref, b2_ref, out_ref):
    a = jnp.dot(ea_ref[...], w0t_ref[...], preferred_element_type=jnp.float32)
    a = _ssp(a + b0_ref[...])
    w = jnp.dot(a, w2t_ref[...], preferred_element_type=jnp.float32) + b2_ref[...]
    el = el_ref[...]
    c = 0.5 * (jnp.cos(el * (PI / CUTOFF)) + 1.0)
    c = c * (el <= CUTOFF).astype(jnp.float32) * (el >= 0.0).astype(jnp.float32)
    out_ref[...] = w * c


def _filter_tc(edge_attr, edge_length, w0t, b0, w2t, b2):
    g = edge_attr.shape[1]
    grid = E // BE_TC
    return pl.pallas_call(
        _filter_body,
        grid=(grid,),
        in_specs=[
            pl.BlockSpec((BE_TC, g), lambda i: (i, 0)),
            pl.BlockSpec((BE_TC, 1), lambda i: (i, 0)),
            pl.BlockSpec((g, F), lambda i: (0, 0)),
            pl.BlockSpec((1, F), lambda i: (0, 0)),
            pl.BlockSpec((F, F), lambda i: (0, 0)),
            pl.BlockSpec((1, F), lambda i: (0, 0)),
        ],
        out_specs=pl.BlockSpec((BE_TC, F), lambda i: (i, 0)),
        out_shape=jax.ShapeDtypeStruct((E, F), jnp.float32),
    )(edge_attr, edge_length, w0t, b0, w2t, b2)


def _lin1_body(x_ref, wt_ref, out_ref):
    out_ref[...] = jnp.dot(x_ref[...], wt_ref[...],
                           preferred_element_type=jnp.float32)


def _lin1_tc(x, lin1t):
    rb = 1000
    return pl.pallas_call(
        _lin1_body,
        grid=(N // rb,),
        in_specs=[
            pl.BlockSpec((rb, F), lambda i: (i, 0)),
            pl.BlockSpec((F, F), lambda i: (0, 0)),
        ],
        out_specs=pl.BlockSpec((rb, F), lambda i: (i, 0)),
        out_shape=jax.ShapeDtypeStruct((N, F), jnp.float32),
    )(x, lin1t)


def _msg_body(h_hbm, w_hbm, src_hbm, dst_hbm, out_hbm,
              src_v, dst_v, rows_v, wv, agg_sh, sem):
    c = lax.axis_index("c")
    s = lax.axis_index("s")
    wid = c * NS + s
    wbase = wid * EPW

    def zrow(i, _):
        for j in range(F // 16):
            rows_v[i, pl.ds(j * 16, 16)] = jnp.zeros((16,), jnp.float32)
        return 0
    lax.fori_loop(0, BE_SC, zrow, 0)

    def zcopy(k, _):
        pltpu.sync_copy(rows_v, agg_sh.at[pl.ds(s * RPT + k * BE_SC, BE_SC)])
        return 0
    lax.fori_loop(0, RPT // BE_SC, zcopy, 0)
    plsc.subcore_barrier()

    def block(b, _):
        base = wbase + b * BE_SC
        pltpu.sync_copy(src_hbm.at[pl.ds(base, BE_SC)], src_v)
        pltpu.sync_copy(dst_hbm.at[pl.ds(base, BE_SC)], dst_v)
        gather = pltpu.async_copy(h_hbm.at[src_v], rows_v, sem)
        pltpu.sync_copy(w_hbm.at[pl.ds(base, BE_SC)], wv)
        gather.wait()

        def mul(i, _):
            for j in range(F // 16):
                sl = pl.ds(j * 16, 16)
                rows_v[i, sl] = rows_v[i, sl] * wv[i, sl]
            return 0
        lax.fori_loop(0, BE_SC, mul, 0)

        pltpu.sync_copy(rows_v, agg_sh.at[dst_v], add=True)
        return 0
    lax.fori_loop(0, NB_SC, block, 0)

    plsc.subcore_barrier()
    pltpu.sync_copy(agg_sh.at[pl.ds(s * RPT, RPT)],
                    out_hbm.at[c, pl.ds(s * RPT, RPT)])


def _msg_sc(h, w, src, dst):
    mesh = plsc.VectorSubcoreMesh(core_axis_name="c", subcore_axis_name="s")
    f = pl.kernel(
        _msg_body,
        out_type=jax.ShapeDtypeStruct((NC, NPAD, F), jnp.float32),
        mesh=mesh,
        scratch_types=[
            pltpu.VMEM((BE_SC,), jnp.int32),
            pltpu.VMEM((BE_SC,), jnp.int32),
            pltpu.VMEM((BE_SC, F), jnp.float32),
            pltpu.VMEM((BE_SC, F), jnp.float32),
            pltpu.VMEM_SHARED((NPAD, F), jnp.float32),
            pltpu.SemaphoreType.DMA,
        ],
    )
    return f(h, w, src, dst)


def _final_body(p_ref, l2t_ref, b2_ref, lt_ref, b_ref, out_ref):
    agg = p_ref[0] + p_ref[1]
    h2 = jnp.dot(agg, l2t_ref[...], preferred_element_type=jnp.float32)
    h2 = _ssp(h2 + b2_ref[...])
    out_ref[...] = jnp.dot(h2, lt_ref[...],
                           preferred_element_type=jnp.float32) + b_ref[...]


def _final_tc(parts, lin2t, b2, lint, b):
    rb = 1000
    out = parts.shape[-1]
    return pl.pallas_call(
        _final_body,
        grid=(N // rb,),
        in_specs=[
            pl.BlockSpec((NC, rb, F), lambda i: (0, i, 0)),
            pl.BlockSpec((F, out), lambda i: (0, 0)),
            pl.BlockSpec((1, out), lambda i: (0, 0)),
            pl.BlockSpec((out, out), lambda i: (0, 0)),
            pl.BlockSpec((1, out), lambda i: (0, 0)),
        ],
        out_specs=pl.BlockSpec((rb, out), lambda i: (i, 0)),
        out_shape=jax.ShapeDtypeStruct((N, out), jnp.float32),
    )(parts, lin2t, b2, lint, b)


@jax.jit
def kernel(x, edge_index, edge_length, edge_attr,
           lin1_w, nnW0_w, nnW0_b, nnW2_w, nnW2_b,
           lin2_w, lin2_b, lin_w, lin_b):
    w = _filter_tc(edge_attr, edge_length[:, None],
                   nnW0_w.T, nnW0_b[None, :], nnW2_w.T, nnW2_b[None, :])
    h = _lin1_tc(x, lin1_w.T)
    src = edge_index[0]
    dst = edge_index[1]
    parts = _msg_sc(h, w, src, dst)
    return _final_tc(parts, lin2_w.T, lin2_b[None, :],
                     lin_w.T, lin_b[None, :])


# speedup vs baseline: 5.4715x; 3.9411x over previous
import functools
import math

import jax
import jax.numpy as jnp
from jax import lax
from jax.experimental import pallas as pl
from jax.experimental.pallas import tpu as pltpu
from jax.experimental.pallas import tpu_sc as plsc

PI = math.pi
CUTOFF = 10.0
LOG2 = math.log(2.0)

N = 10000
E = 320000
F = 128

NC = 2
NS = 16
EPW = E // (NC * NS)
BE_SC = 80
NB_SC = EPW // BE_SC
NPAD = 10240
RPT = NPAD // NS

BE_TC = 3200
CHUNKS = (102400, 102400, 115200)

import numpy as _np
_PERM = _np.empty(F, dtype=_np.int32)
for _j in range(F // 32):
    for _m in range(16):
        _PERM[16 * _j + _m] = 32 * _j + _m
        _PERM[64 + 16 * _j + _m] = 32 * _j + 16 + _m


def _ssp(v):
    return jax.nn.softplus(v) - LOG2


def _filter_body(eat_ref, el_ref, w0t_ref, b0_ref, w2t_ref, b2_ref, out_ref):
    a = lax.dot_general(eat_ref[...], w0t_ref[...],
                        (((0,), (0,)), ((), ())),
                        preferred_element_type=jnp.float32)
    a = _ssp(a + b0_ref[...])
    w = jnp.dot(a, w2t_ref[...], preferred_element_type=jnp.float32) + b2_ref[...]
    el = el_ref[...]
    c = 0.5 * (jnp.cos(el * (PI / CUTOFF)) + 1.0)
    c = c * (el <= CUTOFF).astype(jnp.float32) * (el >= 0.0).astype(jnp.float32)
    out_ref[...] = w * c.reshape(BE_TC, 1)


def _filter_tc(edge_attr_t, edge_length, w0t, b0, w2t, b2, ebase, esz):
    g = edge_attr_t.shape[0]
    ob = ebase // BE_TC
    return pl.pallas_call(
        _filter_body,
        grid=(esz // BE_TC,),
        in_specs=[
            pl.BlockSpec((g, BE_TC), lambda i: (0, i + ob)),
            pl.BlockSpec((1, BE_TC), lambda i: (0, i + ob)),
            pl.BlockSpec((g, F), lambda i: (0, 0)),
            pl.BlockSpec((1, F), lambda i: (0, 0)),
            pl.BlockSpec((F, F), lambda i: (0, 0)),
            pl.BlockSpec((1, F), lambda i: (0, 0)),
        ],
        out_specs=pl.BlockSpec((BE_TC, F), lambda i: (i, 0)),
        out_shape=jax.ShapeDtypeStruct((esz, F), jnp.float32),
    )(edge_attr_t, edge_length, w0t, b0, w2t, b2)


def _lin1_body(x_ref, wt_ref, out_ref):
    out_ref[...] = jnp.dot(x_ref[...], wt_ref[...],
                           preferred_element_type=jnp.float32)


def _lin1_tc(x, lin1t):
    rb = 1000
    return pl.pallas_call(
        _lin1_body,
        grid=(N // rb,),
        in_specs=[
            pl.BlockSpec((rb, F), lambda i: (i, 0)),
            pl.BlockSpec((F, F), lambda i: (0, 0)),
        ],
        out_specs=pl.BlockSpec((rb, F), lambda i: (i, 0)),
        out_shape=jax.ShapeDtypeStruct((N, F), jnp.float32),
    )(x, lin1t)


def _make_msg_body(nb, ebase):
    def _msg_body(h_hbm, w_hbm, src_hbm, dst_hbm, out_hbm,
                  src0, src1, dst0, dst1, rows0, rows1, wv0, wv1, agg_sh,
                  sg0, sg1, sw0, sw1, si0, si1, sd0, sd1, ss0, ss1):
        c = lax.axis_index("c")
        s = lax.axis_index("s")
        wid = c * NS + s
        wbase_w = wid * nb * BE_SC
        wbase_e = ebase + wbase_w

        srcs, dsts = (src0, src1), (dst0, dst1)
        rows, wvs = (rows0, rows1), (wv0, wv1)
        sgs, sws, sis = (sg0, sg1), (sw0, sw1), (si0, si1)
        sds, sss = (sd0, sd1), (ss0, ss1)

        def zrow(i, _):
            for j in range(F // 16):
                rows0[i, pl.ds(j * 16, 16)] = jnp.zeros((16,), jnp.float32)
            return 0
        lax.fori_loop(0, BE_SC, zrow, 0)

        def zcopy(k, _):
            pltpu.sync_copy(rows0, agg_sh.at[pl.ds(s * RPT + k * BE_SC, BE_SC)])
            return 0
        lax.fori_loop(0, RPT // BE_SC, zcopy, 0)
        plsc.subcore_barrier()

        def fetch_src(b, buf):
            pltpu.async_copy(src_hbm.at[pl.ds(wbase_e + b * BE_SC, BE_SC)],
                             srcs[buf], sis[buf])

        def wait_src(buf):
            pltpu.make_async_copy(src_hbm.at[pl.ds(0, BE_SC)],
                                  srcs[buf], sis[buf]).wait()

        def fetch_dst(b, buf):
            pltpu.async_copy(dst_hbm.at[pl.ds(wbase_e + b * BE_SC, BE_SC)],
                             dsts[buf], sds[buf])

        def wait_dst(buf):
            pltpu.make_async_copy(dst_hbm.at[pl.ds(0, BE_SC)],
                                  dsts[buf], sds[buf]).wait()

        def issue(b, buf):
            pltpu.async_copy(h_hbm.at[srcs[buf]], rows[buf], sgs[buf])
            pltpu.async_copy(w_hbm.at[pl.ds(wbase_w + b * BE_SC, BE_SC)],
                             wvs[buf], sws[buf])

        def wait_in(buf):
            pltpu.make_async_copy(h_hbm.at[srcs[buf]], rows[buf], sgs[buf]).wait()
            pltpu.make_async_copy(w_hbm.at[pl.ds(0, BE_SC)],
                                  wvs[buf], sws[buf]).wait()

        def mul(buf):
            def body(i, _):
                for j in range(F // 16):
                    sl = pl.ds(j * 16, 16)
                    rows[buf][i, sl] = rows[buf][i, sl] * wvs[buf][i, sl]
                return 0
            lax.fori_loop(0, BE_SC, body, 0)

        def scatter(buf):
            pltpu.async_copy(rows[buf], agg_sh.at[dsts[buf]], sss[buf], add=True)

        def wait_scatter(buf):
            pltpu.make_async_copy(rows[buf], agg_sh.at[dsts[buf]], sss[buf]).wait()

        fetch_src(0, 0)
        fetch_dst(0, 0)
        wait_src(0)
        issue(0, 0)
        fetch_src(1, 1)

        def pair(i, _):
            for k in range(2):
                b = 2 * i + k
                buf, obuf = k, 1 - k
                @pl.when(jnp.logical_or(i > 0, k > 0))
                def _():
                    wait_scatter(obuf)
                @pl.when(b + 1 < nb)
                def _():
                    fetch_dst(b + 1, obuf)
                    wait_src(obuf)
                    issue(b + 1, obuf)
                wait_in(buf)
                @pl.when(b + 2 < nb)
                def _():
                    fetch_src(b + 2, buf)
                mul(buf)
                wait_dst(buf)
                scatter(buf)
            return 0
        lax.fori_loop(0, nb // 2, pair, 0)

        if nb % 2 == 1:
            wait_in(0)
            mul(0)
            wait_scatter(1)
            wait_dst(0)
            scatter(0)
            wait_scatter(0)
        else:
            wait_scatter(1)

        plsc.subcore_barrier()
        pltpu.sync_copy(agg_sh.at[pl.ds(s * RPT, RPT)],
                        out_hbm.at[c, pl.ds(s * RPT, RPT)])

    return _msg_body


def _msg_sc(h, w, src, dst, nb, ebase):
    mesh = plsc.VectorSubcoreMesh(core_axis_name="c", subcore_axis_name="s")
    f = pl.kernel(
        _make_msg_body(nb, ebase),
        out_type=jax.ShapeDtypeStruct((NC, NPAD, F), jnp.float32),
        mesh=mesh,
        scratch_types=[
            pltpu.VMEM((BE_SC,), jnp.int32),
            pltpu.VMEM((BE_SC,), jnp.int32),
            pltpu.VMEM((BE_SC,), jnp.int32),
            pltpu.VMEM((BE_SC,), jnp.int32),
            pltpu.VMEM((BE_SC, F), jnp.float32),
            pltpu.VMEM((BE_SC, F), jnp.float32),
            pltpu.VMEM((BE_SC, F), jnp.float32),
            pltpu.VMEM((BE_SC, F), jnp.float32),
            pltpu.VMEM_SHARED((NPAD, F), jnp.float32),
        ] + [pltpu.SemaphoreType.DMA] * 10,
    )
    return f(h, w, src, dst)


def _final_body(*refs):
    p_refs, (l2t_ref, b2_ref, lt_ref, b_ref, out_ref) = refs[:-5], refs[-5:]
    agg = sum(p[0] + p[1] for p in p_refs)
    h2 = jnp.dot(agg, l2t_ref[...], preferred_element_type=jnp.float32)
    h2 = _ssp(h2 + b2_ref[...])
    out_ref[...] = jnp.dot(h2, lt_ref[...],
                           preferred_element_type=jnp.float32) + b_ref[...]


def _final_tc(parts, lin2t, b2, lint, b):
    rb = 1000
    out = lint.shape[-1]
    return pl.pallas_call(
        _final_body,
        grid=(N // rb,),
        in_specs=[pl.BlockSpec((NC, rb, F), lambda i: (0, i, 0))
                  for _ in parts] + [
            pl.BlockSpec((F, out), lambda i: (0, 0)),
            pl.BlockSpec((1, out), lambda i: (0, 0)),
            pl.BlockSpec((out, out), lambda i: (0, 0)),
            pl.BlockSpec((1, out), lambda i: (0, 0)),
        ],
        out_specs=pl.BlockSpec((rb, out), lambda i: (i, 0)),
        out_shape=jax.ShapeDtypeStruct((N, out), jnp.float32),
    )(*parts, lin2t, b2, lint, b)


@jax.jit
def kernel(x, edge_index, edge_length, edge_attr,
           lin1_w, nnW0_w, nnW0_b, nnW2_w, nnW2_b,
           lin2_w, lin2_b, lin_w, lin_b):
    h = _lin1_tc(x, lin1_w.T)
    src = edge_index[0]
    dst = edge_index[1]
    eat = edge_attr.T
    el2 = edge_length[None, :]
    w0t, b0 = nnW0_w.T, nnW0_b[None, :]
    w2t, b2 = nnW2_w.T, nnW2_b[None, :]
    parts = []
    off = 0
    for sz in CHUNKS:
        w_c = _filter_tc(eat, el2, w0t, b0, w2t, b2, off, sz)
        parts.append(_msg_sc(h, w_c, src, dst, sz // (NC * NS * BE_SC), off))
        off += sz
    return _final_tc(parts, lin2_w.T, lin2_b[None, :],
                     lin_w.T, lin_b[None, :])
